# baseline (speedup 1.0000x reference)
import functools

import jax
import jax.numpy as jnp
from jax import lax
from jax.experimental import pallas as pl
from jax.experimental.pallas import tpu as pltpu
from jax.experimental.pallas import tpu_sc as plsc

B = 16384
D = 64
BRAND_V = 100000
SMALL_VOCABS = (100, 4, 256, 1000)

_NC, _NS = 2, 16
NW = _NC * _NS
B_PER_W = B // NW
CHUNK = 128
NCHUNK = B_PER_W // CHUNK


def _sc_gather_brand(table, idx_all):
    mesh = plsc.VectorSubcoreMesh(core_axis_name="c", subcore_axis_name="s")

    @functools.partial(
        pl.kernel,
        mesh=mesh,
        out_type=jax.ShapeDtypeStruct((NW, NCHUNK, CHUNK, D), jnp.float32),
        scratch_types=[
            pltpu.VMEM((NCHUNK, CHUNK), jnp.int32),
            pltpu.VMEM((NCHUNK, CHUNK, D), jnp.float32),
            pltpu.SemaphoreType.DMA,
        ],
        compiler_params=pltpu.CompilerParams(use_tc_tiling_on_sc=False),
    )
    def k(tab, idx_hbm, out_hbm, idx_v, rows_v, sem):
        wid = lax.axis_index("s") * _NC + lax.axis_index("c")
        pltpu.sync_copy(idx_hbm.at[wid], idx_v)
        copies = [pltpu.async_copy(tab.at[idx_v.at[j]], rows_v.at[j], sem)
                  for j in range(NCHUNK)]
        for c in copies:
            c.wait()
        pltpu.sync_copy(rows_v, out_hbm.at[wid])

    return k(table, idx_all).reshape(B, D)


def _tower_body(uid_ref, iid_ref, brand_ref,
                a_idx, g_idx, c_idx, k_idx, text_ref,
                ta_ref, tg_ref, tc_ref, tk_ref,
                tpW_ref, tpb_ref,
                uW1_ref, ub1_ref, uW2_ref, ub2_ref, uW3_ref, ub3_ref,
                iW1_ref, ib1_ref, iW2_ref, ib2_ref, iW3_ref, ib3_ref,
                u_out, i_out):
    f32 = jnp.float32
    bt = u_out.shape[0]

    def onehot_rows(idx_ref, tab_ref, vocab):
        idx = idx_ref[0, 0, :]
        oh = (idx[:, None] == lax.broadcasted_iota(jnp.int32, (bt, vocab), 1))
        return jnp.dot(oh.astype(f32), tab_ref[...], preferred_element_type=f32)

    def mlp_from_blocks(blocks, W1, b1, W2, b2, W3, b3):
        acc = b1
        for f, blk in enumerate(blocks):
            acc = acc + jnp.dot(blk, W1[64 * f:64 * (f + 1), :],
                                preferred_element_type=f32)
        h = jnp.maximum(acc, 0.0)
        h = jnp.maximum(jnp.dot(h, W2, preferred_element_type=f32) + b2, 0.0)
        return jnp.dot(h, W3, preferred_element_type=f32) + b3

    def l2norm(x):
        n = jnp.sqrt(jnp.sum(x * x, axis=1, keepdims=True))
        return x / jnp.maximum(n, 1e-12)

    u_blocks = [
        uid_ref[...].astype(f32),
        onehot_rows(a_idx, ta_ref, SMALL_VOCABS[0]),
        onehot_rows(g_idx, tg_ref, SMALL_VOCABS[1]),
        onehot_rows(c_idx, tc_ref, SMALL_VOCABS[2]),
    ]
    u = mlp_from_blocks(u_blocks, uW1_ref[...], ub1_ref[...],
                        uW2_ref[...], ub2_ref[...], uW3_ref[...], ub3_ref[...])
    u_out[...] = l2norm(u)

    tp = jnp.dot(text_ref[...], tpW_ref[...],
                 preferred_element_type=f32) + tpb_ref[...]
    i_blocks = [
        iid_ref[...].astype(f32),
        onehot_rows(k_idx, tk_ref, SMALL_VOCABS[3]),
        brand_ref[...],
        tp,
    ]
    it = mlp_from_blocks(i_blocks, iW1_ref[...], ib1_ref[...],
                         iW2_ref[...], ib2_ref[...], iW3_ref[...], ib3_ref[...])
    i_out[...] = l2norm(it)


def _towers_tc(uid_rows, iid_rows, brand_rows,
               a_idx, g_idx, c_idx, k_idx, text,
               ta, tg, tc, tk, tpW, tpb,
               uW1, ub1, uW2, ub2, uW3, ub3,
               iW1, ib1, iW2, ib2, iW3, ib3, bt=512):
    grid = (B // bt,)
    full = lambda shape: pl.BlockSpec(shape, lambda i: (0,) * len(shape))
    row_spec = pl.BlockSpec((bt, D), lambda i: (i, 0))
    idx_spec = pl.BlockSpec((1, 1, bt), lambda i: (i, 0, 0))
    in_specs = [
        row_spec, row_spec, row_spec,
        idx_spec, idx_spec, idx_spec, idx_spec,
        pl.BlockSpec((bt, 768), lambda i: (i, 0)),
        full((SMALL_VOCABS[0], D)), full((SMALL_VOCABS[1], D)),
        full((SMALL_VOCABS[2], D)), full((SMALL_VOCABS[3], D)),
        full((768, D)), full((1, D)),
        full((256, 256)), full((1, 256)),
        full((256, 128)), full((1, 128)),
        full((128, D)), full((1, D)),
        full((256, 256)), full((1, 256)),
        full((256, 128)), full((1, 128)),
        full((128, D)), full((1, D)),
    ]
    out_specs = (row_spec, row_spec)
    return pl.pallas_call(
        _tower_body,
        grid=grid,
        in_specs=in_specs,
        out_specs=out_specs,
        out_shape=(jax.ShapeDtypeStruct((B, D), jnp.float32),
                   jax.ShapeDtypeStruct((B, D), jnp.float32)),
        compiler_params=pltpu.CompilerParams(
            dimension_semantics=("arbitrary",)),
    )(uid_rows, iid_rows, brand_rows, a_idx, g_idx, c_idx, k_idx, text,
      ta, tg, tc, tk, tpW, tpb,
      uW1, ub1, uW2, ub2, uW3, ub3, iW1, ib1, iW2, ib2, iW3, ib3)


def kernel(user_user_id, user_age_bucket, user_gender, user_country,
           item_item_id, item_category, item_brand, text_features,
           emb_user_user_id, emb_user_age_bucket, emb_user_gender,
           emb_user_country, emb_item_item_id, emb_item_category,
           emb_item_brand, text_proj_W, text_proj_b,
           uW1, ub1, uW2, ub2, uW3, ub3, iW1, ib1, iW2, ib2, iW3, ib3):
    emb_user_b, _ = lax.optimization_barrier(
        (emb_user_user_id.astype(jnp.bfloat16), user_user_id))
    emb_item_b, _ = lax.optimization_barrier(
        (emb_item_item_id.astype(jnp.bfloat16), user_user_id))
    uid_rows = jnp.take(emb_user_b, user_user_id, axis=0, mode="clip")
    iid_rows = jnp.take(emb_item_b, item_item_id, axis=0, mode="clip")

    brand_rows = _sc_gather_brand(
        emb_item_brand,
        item_brand.astype(jnp.int32).reshape(NW, NCHUNK, CHUNK))

    small_idx = [ix.astype(jnp.int32).reshape(B // 512, 1, 512)
                 for ix in (user_age_bucket, user_gender, user_country,
                            item_category)]
    u, it = _towers_tc(
        uid_rows, iid_rows, brand_rows, *small_idx, text_features,
        emb_user_age_bucket, emb_user_gender, emb_user_country,
        emb_item_category,
        text_proj_W, text_proj_b.reshape(1, D),
        uW1, ub1.reshape(1, 256), uW2, ub2.reshape(1, 128),
        uW3, ub3.reshape(1, D),
        iW1, ib1.reshape(1, 256), iW2, ib2.reshape(1, 128),
        iW3, ib3.reshape(1, D))
    return (u, it)


# speedup vs baseline: 1.2025x; 1.2025x over previous
import functools

import jax
import jax.numpy as jnp
from jax import lax
from jax.experimental import pallas as pl
from jax.experimental.pallas import tpu as pltpu
from jax.experimental.pallas import tpu_sc as plsc

B = 16384
D = 64
BRAND_V = 100000
SMALL_VOCABS = (100, 4, 256, 1000)

_NC, _NS = 2, 16
NW = _NC * _NS
B_PER_W = B // NW
CHUNK = 128
NCHUNK = B_PER_W // CHUNK


def _sc_gather_brand(table, idx_all):
    mesh = plsc.VectorSubcoreMesh(core_axis_name="c", subcore_axis_name="s")

    @functools.partial(
        pl.kernel,
        mesh=mesh,
        out_type=jax.ShapeDtypeStruct((NW, NCHUNK, CHUNK, D), jnp.float32),
        scratch_types=[
            pltpu.VMEM((NCHUNK, CHUNK), jnp.int32),
            pltpu.VMEM((NCHUNK, CHUNK, D), jnp.float32),
            pltpu.SemaphoreType.DMA,
        ],
        compiler_params=pltpu.CompilerParams(use_tc_tiling_on_sc=False),
    )
    def k(tab, idx_hbm, out_hbm, idx_v, rows_v, sem):
        wid = lax.axis_index("s") * _NC + lax.axis_index("c")
        pltpu.sync_copy(idx_hbm.at[wid], idx_v)
        copies = [pltpu.async_copy(tab.at[idx_v.at[j]], rows_v.at[j], sem)
                  for j in range(NCHUNK)]
        for c in copies:
            c.wait()
        pltpu.sync_copy(rows_v, out_hbm.at[wid])

    return k(table, idx_all).reshape(B, D)


def _tower_body(uid_ref, iid_ref, brand_ref,
                a_idx, g_idx, c_idx, k_idx, text_ref,
                ta_ref, tg_ref, tc_ref, tk_ref,
                tpW_ref, tpb_ref,
                uW1_ref, ub1_ref, uW2_ref, ub2_ref, uW3_ref, ub3_ref,
                iW1_ref, ib1_ref, iW2_ref, ib2_ref, iW3_ref, ib3_ref,
                u_out, i_out):
    f32 = jnp.float32
    bt = u_out.shape[0]

    def onehot_rows(idx_ref, tab_ref, vocab):
        idx = idx_ref[0, 0, :]
        oh = (idx[:, None] == lax.broadcasted_iota(jnp.int32, (bt, vocab), 1))
        return jnp.dot(oh.astype(f32), tab_ref[...], preferred_element_type=f32)

    def mlp_from_blocks(blocks, W1, b1, W2, b2, W3, b3):
        acc = b1
        for f, blk in enumerate(blocks):
            acc = acc + jnp.dot(blk, W1[64 * f:64 * (f + 1), :],
                                preferred_element_type=f32)
        h = jnp.maximum(acc, 0.0)
        h = jnp.maximum(jnp.dot(h, W2, preferred_element_type=f32) + b2, 0.0)
        return jnp.dot(h, W3, preferred_element_type=f32) + b3

    def l2norm(x):
        n = jnp.sqrt(jnp.sum(x * x, axis=1, keepdims=True))
        return x / jnp.maximum(n, 1e-12)

    u_blocks = [
        uid_ref[...].astype(f32),
        onehot_rows(a_idx, ta_ref, SMALL_VOCABS[0]),
        onehot_rows(g_idx, tg_ref, SMALL_VOCABS[1]),
        onehot_rows(c_idx, tc_ref, SMALL_VOCABS[2]),
    ]
    u = mlp_from_blocks(u_blocks, uW1_ref[...], ub1_ref[...],
                        uW2_ref[...], ub2_ref[...], uW3_ref[...], ub3_ref[...])
    u_out[...] = l2norm(u)

    tp = jnp.dot(text_ref[...], tpW_ref[...],
                 preferred_element_type=f32) + tpb_ref[...]
    i_blocks = [
        iid_ref[...].astype(f32),
        onehot_rows(k_idx, tk_ref, SMALL_VOCABS[3]),
        brand_ref[...],
        tp,
    ]
    it = mlp_from_blocks(i_blocks, iW1_ref[...], ib1_ref[...],
                         iW2_ref[...], ib2_ref[...], iW3_ref[...], ib3_ref[...])
    i_out[...] = l2norm(it)


def _towers_tc(uid_rows, iid_rows, brand_rows,
               a_idx, g_idx, c_idx, k_idx, text,
               ta, tg, tc, tk, tpW, tpb,
               uW1, ub1, uW2, ub2, uW3, ub3,
               iW1, ib1, iW2, ib2, iW3, ib3, bt=512):
    grid = (B // bt,)
    full = lambda shape: pl.BlockSpec(shape, lambda i: (0,) * len(shape))
    row_spec = pl.BlockSpec((bt, D), lambda i: (i, 0))
    idx_spec = pl.BlockSpec((1, 1, bt), lambda i: (i, 0, 0))
    in_specs = [
        row_spec, row_spec, row_spec,
        idx_spec, idx_spec, idx_spec, idx_spec,
        pl.BlockSpec((bt, 768), lambda i: (i, 0)),
        full((SMALL_VOCABS[0], D)), full((SMALL_VOCABS[1], D)),
        full((SMALL_VOCABS[2], D)), full((SMALL_VOCABS[3], D)),
        full((768, D)), full((1, D)),
        full((256, 256)), full((1, 256)),
        full((256, 128)), full((1, 128)),
        full((128, D)), full((1, D)),
        full((256, 256)), full((1, 256)),
        full((256, 128)), full((1, 128)),
        full((128, D)), full((1, D)),
    ]
    out_specs = (row_spec, row_spec)
    return pl.pallas_call(
        _tower_body,
        grid=grid,
        in_specs=in_specs,
        out_specs=out_specs,
        out_shape=(jax.ShapeDtypeStruct((B, D), jnp.float32),
                   jax.ShapeDtypeStruct((B, D), jnp.float32)),
        compiler_params=pltpu.CompilerParams(
            dimension_semantics=("arbitrary",)),
    )(uid_rows, iid_rows, brand_rows, a_idx, g_idx, c_idx, k_idx, text,
      ta, tg, tc, tk, tpW, tpb,
      uW1, ub1, uW2, ub2, uW3, ub3, iW1, ib1, iW2, ib2, iW3, ib3)


def kernel(user_user_id, user_age_bucket, user_gender, user_country,
           item_item_id, item_category, item_brand, text_features,
           emb_user_user_id, emb_user_age_bucket, emb_user_gender,
           emb_user_country, emb_item_item_id, emb_item_category,
           emb_item_brand, text_proj_W, text_proj_b,
           uW1, ub1, uW2, ub2, uW3, ub3, iW1, ib1, iW2, ib2, iW3, ib3):
    uid_rows = jnp.take(emb_user_user_id, user_user_id, axis=0, mode="clip")
    iid_rows = jnp.take(emb_item_item_id, item_item_id, axis=0, mode="clip")

    brand_rows = _sc_gather_brand(
        emb_item_brand,
        item_brand.astype(jnp.int32).reshape(NW, NCHUNK, CHUNK))

    small_idx = [ix.astype(jnp.int32).reshape(B // 512, 1, 512)
                 for ix in (user_age_bucket, user_gender, user_country,
                            item_category)]
    u, it = _towers_tc(
        uid_rows, iid_rows, brand_rows, *small_idx, text_features,
        emb_user_age_bucket, emb_user_gender, emb_user_country,
        emb_item_category,
        text_proj_W, text_proj_b.reshape(1, D),
        uW1, ub1.reshape(1, 256), uW2, ub2.reshape(1, 128),
        uW3, ub3.reshape(1, D),
        iW1, ib1.reshape(1, 256), iW2, ib2.reshape(1, 128),
        iW3, ib3.reshape(1, D))
    return (u, it)


# speedup vs baseline: 1.2272x; 1.0205x over previous
import functools

import jax
import jax.numpy as jnp
from jax import lax
from jax.experimental import pallas as pl
from jax.experimental.pallas import tpu as pltpu
from jax.experimental.pallas import tpu_sc as plsc

B = 16384
D = 64
BRAND_V = 100000
SMALL_VOCABS = (100, 4, 256, 1000)

_NC, _NS = 2, 16
NW = _NC * _NS
B_PER_W = B // NW
CHUNK = 128
NCHUNK = B_PER_W // CHUNK


def _sc_gather_brand(table, idx_all):
    mesh = plsc.VectorSubcoreMesh(core_axis_name="c", subcore_axis_name="s")

    @functools.partial(
        pl.kernel,
        mesh=mesh,
        out_type=jax.ShapeDtypeStruct((NW, NCHUNK, CHUNK, D), jnp.float32),
        scratch_types=[
            pltpu.VMEM((NCHUNK, CHUNK), jnp.int32),
            pltpu.VMEM((NCHUNK, CHUNK, D), jnp.float32),
            pltpu.SemaphoreType.DMA,
        ],
        compiler_params=pltpu.CompilerParams(use_tc_tiling_on_sc=False),
    )
    def k(tab, idx_hbm, out_hbm, idx_v, rows_v, sem):
        wid = lax.axis_index("s") * _NC + lax.axis_index("c")
        pltpu.sync_copy(idx_hbm.at[wid], idx_v)
        copies = [pltpu.async_copy(tab.at[idx_v.at[j]], rows_v.at[j], sem)
                  for j in range(NCHUNK)]
        for c in copies:
            c.wait()
        pltpu.sync_copy(rows_v, out_hbm.at[wid])

    return k(table, idx_all).reshape(B, D)


def _tower_body(uid_ref, iid_ref, brand_ref,
                a_idx, g_idx, c_idx, k_idx, text_ref,
                ta_ref, tg_ref, tc_ref, tk_ref,
                tpW_ref, tpb_ref,
                uW1_ref, ub1_ref, uW2_ref, ub2_ref, uW3_ref, ub3_ref,
                iW1_ref, ib1_ref, iW2_ref, ib2_ref, iW3_ref, ib3_ref,
                u_out, i_out):
    f32 = jnp.float32
    bt = u_out.shape[0]

    def onehot_rows(idx_ref, tab_ref, vocab):
        idx = idx_ref[0, 0, :]
        oh = (idx[:, None] == lax.broadcasted_iota(jnp.int32, (bt, vocab), 1))
        return jnp.dot(oh.astype(f32), tab_ref[...], preferred_element_type=f32)

    def mlp_from_blocks(blocks, W1, b1, W2, b2, W3, b3):
        acc = b1
        for f, blk in enumerate(blocks):
            acc = acc + jnp.dot(blk, W1[64 * f:64 * (f + 1), :],
                                preferred_element_type=f32)
        h = jnp.maximum(acc, 0.0)
        h = jnp.maximum(jnp.dot(h, W2, preferred_element_type=f32) + b2, 0.0)
        return jnp.dot(h, W3, preferred_element_type=f32) + b3

    def l2norm(x):
        n = jnp.sqrt(jnp.sum(x * x, axis=1, keepdims=True))
        return x / jnp.maximum(n, 1e-12)

    u_blocks = [
        uid_ref[...].astype(f32),
        onehot_rows(a_idx, ta_ref, SMALL_VOCABS[0]),
        onehot_rows(g_idx, tg_ref, SMALL_VOCABS[1]),
        onehot_rows(c_idx, tc_ref, SMALL_VOCABS[2]),
    ]
    u = mlp_from_blocks(u_blocks, uW1_ref[...], ub1_ref[...],
                        uW2_ref[...], ub2_ref[...], uW3_ref[...], ub3_ref[...])
    u_out[...] = l2norm(u)

    tp = jnp.dot(text_ref[...], tpW_ref[...],
                 preferred_element_type=f32) + tpb_ref[...]
    i_blocks = [
        iid_ref[...].astype(f32),
        onehot_rows(k_idx, tk_ref, SMALL_VOCABS[3]),
        brand_ref[...],
        tp,
    ]
    it = mlp_from_blocks(i_blocks, iW1_ref[...], ib1_ref[...],
                         iW2_ref[...], ib2_ref[...], iW3_ref[...], ib3_ref[...])
    i_out[...] = l2norm(it)


def _towers_tc(uid_rows, iid_rows, brand_rows,
               a_idx, g_idx, c_idx, k_idx, text,
               ta, tg, tc, tk, tpW, tpb,
               uW1, ub1, uW2, ub2, uW3, ub3,
               iW1, ib1, iW2, ib2, iW3, ib3, bt=1024):
    grid = (B // bt,)
    full = lambda shape: pl.BlockSpec(shape, lambda i: (0,) * len(shape))
    row_spec = pl.BlockSpec((bt, D), lambda i: (i, 0))
    idx_spec = pl.BlockSpec((1, 1, bt), lambda i: (i, 0, 0))
    in_specs = [
        row_spec, row_spec, row_spec,
        idx_spec, idx_spec, idx_spec, idx_spec,
        pl.BlockSpec((bt, 768), lambda i: (i, 0)),
        full((SMALL_VOCABS[0], D)), full((SMALL_VOCABS[1], D)),
        full((SMALL_VOCABS[2], D)), full((SMALL_VOCABS[3], D)),
        full((768, D)), full((1, D)),
        full((256, 256)), full((1, 256)),
        full((256, 128)), full((1, 128)),
        full((128, D)), full((1, D)),
        full((256, 256)), full((1, 256)),
        full((256, 128)), full((1, 128)),
        full((128, D)), full((1, D)),
    ]
    out_specs = (row_spec, row_spec)
    return pl.pallas_call(
        _tower_body,
        grid=grid,
        in_specs=in_specs,
        out_specs=out_specs,
        out_shape=(jax.ShapeDtypeStruct((B, D), jnp.float32),
                   jax.ShapeDtypeStruct((B, D), jnp.float32)),
        compiler_params=pltpu.CompilerParams(
            dimension_semantics=("arbitrary",)),
    )(uid_rows, iid_rows, brand_rows, a_idx, g_idx, c_idx, k_idx, text,
      ta, tg, tc, tk, tpW, tpb,
      uW1, ub1, uW2, ub2, uW3, ub3, iW1, ib1, iW2, ib2, iW3, ib3)


def kernel(user_user_id, user_age_bucket, user_gender, user_country,
           item_item_id, item_category, item_brand, text_features,
           emb_user_user_id, emb_user_age_bucket, emb_user_gender,
           emb_user_country, emb_item_item_id, emb_item_category,
           emb_item_brand, text_proj_W, text_proj_b,
           uW1, ub1, uW2, ub2, uW3, ub3, iW1, ib1, iW2, ib2, iW3, ib3):
    uid_rows = jnp.take(emb_user_user_id, user_user_id, axis=0, mode="clip")
    iid_rows = jnp.take(emb_item_item_id, item_item_id, axis=0, mode="clip")

    brand_rows = _sc_gather_brand(
        emb_item_brand,
        item_brand.astype(jnp.int32).reshape(NW, NCHUNK, CHUNK))

    small_idx = [ix.astype(jnp.int32).reshape(B // 1024, 1, 1024)
                 for ix in (user_age_bucket, user_gender, user_country,
                            item_category)]
    u, it = _towers_tc(
        uid_rows, iid_rows, brand_rows, *small_idx, text_features,
        emb_user_age_bucket, emb_user_gender, emb_user_country,
        emb_item_category,
        text_proj_W, text_proj_b.reshape(1, D),
        uW1, ub1.reshape(1, 256), uW2, ub2.reshape(1, 128),
        uW3, ub3.reshape(1, D),
        iW1, ib1.reshape(1, 256), iW2, ib2.reshape(1, 128),
        iW3, ib3.reshape(1, D))
    return (u, it)


# speedup vs baseline: 1.2309x; 1.0030x over previous
import functools

import jax
import jax.numpy as jnp
from jax import lax
from jax.experimental import pallas as pl
from jax.experimental.pallas import tpu as pltpu
from jax.experimental.pallas import tpu_sc as plsc

B = 16384
D = 64
BRAND_V = 100000
SMALL_VOCABS = (100, 4, 256, 1000)

_NC, _NS = 2, 16
NW = _NC * _NS
B_PER_W = B // NW
CHUNK = 128
NCHUNK = B_PER_W // CHUNK


def _sc_gather_brand(table, idx_all):
    mesh = plsc.VectorSubcoreMesh(core_axis_name="c", subcore_axis_name="s")

    @functools.partial(
        pl.kernel,
        mesh=mesh,
        out_type=jax.ShapeDtypeStruct((NW, NCHUNK, CHUNK, D), jnp.float32),
        scratch_types=[
            pltpu.VMEM((NCHUNK, CHUNK), jnp.int32),
            pltpu.VMEM((NCHUNK, CHUNK, D), jnp.float32),
            pltpu.SemaphoreType.DMA,
        ],
        compiler_params=pltpu.CompilerParams(use_tc_tiling_on_sc=False),
    )
    def k(tab, idx_hbm, out_hbm, idx_v, rows_v, sem):
        wid = lax.axis_index("s") * _NC + lax.axis_index("c")
        pltpu.sync_copy(idx_hbm.at[wid], idx_v)
        copies = [pltpu.async_copy(tab.at[idx_v.at[j]], rows_v.at[j], sem)
                  for j in range(NCHUNK)]
        for c in copies:
            c.wait()
        pltpu.sync_copy(rows_v, out_hbm.at[wid])

    return k(table, idx_all).reshape(B, D)


def _tower_body(uid_ref, iid_ref, brand_ref,
                a_idx, g_idx, c_idx, k_idx, text_ref,
                ta_ref, tg_ref, tc_ref, tk_ref,
                tpW_ref, tpb_ref,
                uW1_ref, ub1_ref, uW2_ref, ub2_ref, uW3_ref, ub3_ref,
                iW1_ref, ib1_ref, iW2_ref, ib2_ref, iW3_ref, ib3_ref,
                u_out, i_out):
    f32 = jnp.float32
    bt = u_out.shape[0]

    def onehot_rows(idx_ref, tab_ref, vocab):
        idx = idx_ref[0, 0, :]
        oh = (idx[:, None] == lax.broadcasted_iota(jnp.int32, (bt, vocab), 1))
        return jnp.dot(oh.astype(f32), tab_ref[...], preferred_element_type=f32)

    def mlp_from_blocks(blocks, W1, b1, W2, b2, W3, b3):
        acc = b1
        for f, blk in enumerate(blocks):
            acc = acc + jnp.dot(blk, W1[64 * f:64 * (f + 1), :],
                                preferred_element_type=f32)
        h = jnp.maximum(acc, 0.0)
        h = jnp.maximum(jnp.dot(h, W2, preferred_element_type=f32) + b2, 0.0)
        return jnp.dot(h, W3, preferred_element_type=f32) + b3

    def l2norm(x):
        n = jnp.sqrt(jnp.sum(x * x, axis=1, keepdims=True))
        return x / jnp.maximum(n, 1e-12)

    u_blocks = [
        uid_ref[...].astype(f32),
        onehot_rows(a_idx, ta_ref, SMALL_VOCABS[0]),
        onehot_rows(g_idx, tg_ref, SMALL_VOCABS[1]),
        onehot_rows(c_idx, tc_ref, SMALL_VOCABS[2]),
    ]
    u = mlp_from_blocks(u_blocks, uW1_ref[...], ub1_ref[...],
                        uW2_ref[...], ub2_ref[...], uW3_ref[...], ub3_ref[...])
    u_out[...] = l2norm(u)

    tp = jnp.dot(text_ref[...], tpW_ref[...],
                 preferred_element_type=f32) + tpb_ref[...]
    i_blocks = [
        iid_ref[...].astype(f32),
        onehot_rows(k_idx, tk_ref, SMALL_VOCABS[3]),
        brand_ref[...],
        tp,
    ]
    it = mlp_from_blocks(i_blocks, iW1_ref[...], ib1_ref[...],
                         iW2_ref[...], ib2_ref[...], iW3_ref[...], ib3_ref[...])
    i_out[...] = l2norm(it)


def _towers_tc(uid_rows, iid_rows, brand_rows,
               a_idx, g_idx, c_idx, k_idx, text,
               ta, tg, tc, tk, tpW, tpb,
               uW1, ub1, uW2, ub2, uW3, ub3,
               iW1, ib1, iW2, ib2, iW3, ib3, bt=2048):
    grid = (B // bt,)
    full = lambda shape: pl.BlockSpec(shape, lambda i: (0,) * len(shape))
    row_spec = pl.BlockSpec((bt, D), lambda i: (i, 0))
    idx_spec = pl.BlockSpec((1, 1, bt), lambda i: (i, 0, 0))
    in_specs = [
        row_spec, row_spec, row_spec,
        idx_spec, idx_spec, idx_spec, idx_spec,
        pl.BlockSpec((bt, 768), lambda i: (i, 0)),
        full((SMALL_VOCABS[0], D)), full((SMALL_VOCABS[1], D)),
        full((SMALL_VOCABS[2], D)), full((SMALL_VOCABS[3], D)),
        full((768, D)), full((1, D)),
        full((256, 256)), full((1, 256)),
        full((256, 128)), full((1, 128)),
        full((128, D)), full((1, D)),
        full((256, 256)), full((1, 256)),
        full((256, 128)), full((1, 128)),
        full((128, D)), full((1, D)),
    ]
    out_specs = (row_spec, row_spec)
    return pl.pallas_call(
        _tower_body,
        grid=grid,
        in_specs=in_specs,
        out_specs=out_specs,
        out_shape=(jax.ShapeDtypeStruct((B, D), jnp.float32),
                   jax.ShapeDtypeStruct((B, D), jnp.float32)),
        compiler_params=pltpu.CompilerParams(
            dimension_semantics=("arbitrary",)),
    )(uid_rows, iid_rows, brand_rows, a_idx, g_idx, c_idx, k_idx, text,
      ta, tg, tc, tk, tpW, tpb,
      uW1, ub1, uW2, ub2, uW3, ub3, iW1, ib1, iW2, ib2, iW3, ib3)


def kernel(user_user_id, user_age_bucket, user_gender, user_country,
           item_item_id, item_category, item_brand, text_features,
           emb_user_user_id, emb_user_age_bucket, emb_user_gender,
           emb_user_country, emb_item_item_id, emb_item_category,
           emb_item_brand, text_proj_W, text_proj_b,
           uW1, ub1, uW2, ub2, uW3, ub3, iW1, ib1, iW2, ib2, iW3, ib3):
    uid_rows = jnp.take(emb_user_user_id, user_user_id, axis=0, mode="clip")
    iid_rows = jnp.take(emb_item_item_id, item_item_id, axis=0, mode="clip")

    brand_rows = _sc_gather_brand(
        emb_item_brand,
        item_brand.astype(jnp.int32).reshape(NW, NCHUNK, CHUNK))

    small_idx = [ix.astype(jnp.int32).reshape(B // 2048, 1, 2048)
                 for ix in (user_age_bucket, user_gender, user_country,
                            item_category)]
    u, it = _towers_tc(
        uid_rows, iid_rows, brand_rows, *small_idx, text_features,
        emb_user_age_bucket, emb_user_gender, emb_user_country,
        emb_item_category,
        text_proj_W, text_proj_b.reshape(1, D),
        uW1, ub1.reshape(1, 256), uW2, ub2.reshape(1, 128),
        uW3, ub3.reshape(1, D),
        iW1, ib1.reshape(1, 256), iW2, ib2.reshape(1, 128),
        iW3, ib3.reshape(1, D))
    return (u, it)
